# baseline (speedup 1.0000x reference)
import jax
import jax.numpy as jnp
from jax import lax
from jax.experimental import pallas as pl
from jax.experimental.pallas import tpu as pltpu
from jax.experimental.pallas import tpu_sc as plsc

B = 16384
L = 200
E = 18
EP = 24
NC = 2
NS = 16
NW = NC * NS
RPW = B // NW
UNIT = 4
UENT = UNIT * L
BLK = 64
UPB = BLK // UNIT
NBLK = RPW // BLK
SH = 256
FLAT = RPW * E


def _phase_uid(w, uid72, uid_g, uid_out, ubuf, remv, sem):
    pltpu.sync_copy(uid_g.at[w], remv)
    pltpu.async_copy(uid72.at[remv], ubuf, sem).wait()
    pltpu.sync_copy(ubuf, uid_out.at[w])


def _phase_singles(w, mid_t, cat_t, mid_idx, cat_idx, mid_out, cat_out,
                   sbuf_a, remv, acc, sem):
    def pack_single(src):
        def row(r, carry):
            v0 = src[r, pl.ds(0, 16)]
            v1 = src[r, pl.ds(2, 16)]
            acc[pl.ds(18 * r, 16)] = v0
            acc[pl.ds(18 * r + 2, 16)] = v1
            return carry
        lax.fori_loop(0, RPW, row, 0)

    for tbl, idx_hbm, out in ((mid_t, mid_idx, mid_out),
                              (cat_t, cat_idx, cat_out)):
        pltpu.sync_copy(idx_hbm.at[w], remv)
        pltpu.async_copy(tbl.at[remv], sbuf_a, sem).wait()
        pack_single(sbuf_a)
        pltpu.sync_copy(acc.at[pl.ds(0, FLAT)], out.at[w])


def _phase_sim(w, cat_t, id1_idx, id2_idx, sim_out,
               sbuf_a, sbuf_b, remv, idxblk, acc, sem):
    pltpu.sync_copy(id1_idx.at[w], remv)
    cp1 = pltpu.async_copy(cat_t.at[remv], sbuf_a, sem)
    pltpu.sync_copy(id2_idx.at[w], idxblk.at[0, pl.ds(0, RPW)])
    cp2 = pltpu.async_copy(cat_t.at[idxblk.at[0, pl.ds(0, RPW)]], sbuf_b, sem)
    cp1.wait()
    cp2.wait()

    def sim_row(r, carry):
        a0 = sbuf_a[r, pl.ds(0, 16)]
        b0 = sbuf_b[r, pl.ds(0, 16)]
        a1 = sbuf_a[r, pl.ds(2, 16)]
        b1 = sbuf_b[r, pl.ds(2, 16)]
        acc[pl.ds(18 * r, 16)] = a0 + b0 + a0 * b0
        acc[pl.ds(18 * r + 2, 16)] = a1 + b1 + a1 * b1
        return carry

    lax.fori_loop(0, RPW, sim_row, 0)
    pltpu.sync_copy(acc.at[pl.ds(0, FLAT)], sim_out.at[w])


def _phase_his(w, mid_t, cat_t, mid_his, cat_his, hmid_out, hcat_out,
               idxblk, hrows, acc, sem):
    z16 = jnp.zeros((16,), jnp.float32)
    for tbl, his_hbm, out in ((mid_t, mid_his, hmid_out),
                              (cat_t, cat_his, hcat_out)):
        def block_body(blk, bcarry, tbl=tbl, his_hbm=his_hbm):
            pltpu.sync_copy(his_hbm.at[w, blk], idxblk)
            pltpu.async_copy(tbl.at[idxblk.at[0]], hrows.at[0], sem)

            def unit_pair(u2, carry, tbl=tbl, blk=blk):
                for p in (0, 1):
                    u = 2 * u2 + p
                    pltpu.make_async_copy(tbl.at[idxblk.at[u]],
                                          hrows.at[p], sem).wait()

                    @pl.when(u + 1 < UPB)
                    def _(tbl=tbl, u=u, p=p):
                        pltpu.async_copy(tbl.at[idxblk.at[u + 1]],
                                         hrows.at[1 - p], sem)

                    for q in range(UNIT):
                        def accum(l8, ab, p=p, q=q):
                            a, b = ab
                            base = q * L + l8 * 8
                            lo = [hrows[p, base + i, pl.ds(0, 16)]
                                  for i in range(8)]
                            hi = [hrows[p, base + i, pl.ds(8, 16)]
                                  for i in range(8)]
                            lo = [lo[0] + lo[1], lo[2] + lo[3],
                                  lo[4] + lo[5], lo[6] + lo[7]]
                            hi = [hi[0] + hi[1], hi[2] + hi[3],
                                  hi[4] + hi[5], hi[6] + hi[7]]
                            lo = [lo[0] + lo[1], lo[2] + lo[3]]
                            hi = [hi[0] + hi[1], hi[2] + hi[3]]
                            return (a + (lo[0] + lo[1]),
                                    b + (hi[0] + hi[1]))
                        a, b = lax.fori_loop(0, L // 8, accum, (z16, z16))
                        r = blk * BLK + u * UNIT + q
                        acc[pl.ds(18 * r, 16)] = a
                        acc[pl.ds(18 * r + 8, 16)] = b
                return carry

            lax.fori_loop(0, UPB // 2, unit_pair, 0)
            return bcarry

        lax.fori_loop(0, NBLK, block_body, 0)
        pltpu.sync_copy(acc.at[pl.ds(0, FLAT)], out.at[w])


def _sc_body(uid72, mid_t, cat_t,
             uid_g, mid_idx, cat_idx, id1_idx, id2_idx,
             mid_his, cat_his,
             uid_out, mid_out, cat_out, sim_out, hmid_out, hcat_out,
             ubuf, sbuf_a, sbuf_b, remv, idxblk, hrows, acc, sem):
    c = lax.axis_index("c")
    s = lax.axis_index("s")
    w = s * NC + c
    _phase_uid(w, uid72, uid_g, uid_out, ubuf, remv, sem)
    _phase_singles(w, mid_t, cat_t, mid_idx, cat_idx, mid_out, cat_out,
                   sbuf_a, remv, acc, sem)
    _phase_sim(w, cat_t, id1_idx, id2_idx, sim_out,
               sbuf_a, sbuf_b, remv, idxblk, acc, sem)
    _phase_his(w, mid_t, cat_t, mid_his, cat_his, hmid_out, hcat_out,
               idxblk, hrows, acc, sem)


def _mlp_body(u72_ref, rem_ref, m_ref, c_ref, hm_ref, hc_ref,
              w1u, w1m, w1c, w1hm, w1hc, b1, w2, b2, w3, b3, y_ref):
    f32 = jnp.float32
    u72 = u72_ref[...]
    rem = rem_ref[...]
    u = u72[:, 0:18]
    for cshift in (18, 36, 54):
        u = jnp.where(rem == cshift, u72[:, cshift:cshift + 18], u)
    h = (jnp.dot(u, w1u[...], preferred_element_type=f32)
         + jnp.dot(m_ref[...], w1m[...], preferred_element_type=f32)
         + jnp.dot(c_ref[...], w1c[...], preferred_element_type=f32)
         + jnp.dot(hm_ref[...], w1hm[...], preferred_element_type=f32)
         + jnp.dot(hc_ref[...], w1hc[...], preferred_element_type=f32)
         + b1[...])
    h = jnp.maximum(h, 0.0)
    h = jnp.dot(h, w2[...], preferred_element_type=f32) + b2[...]
    h = jnp.maximum(h, 0.0)
    y_ref[...] = jnp.dot(h, w3[...], preferred_element_type=f32) + b3[...]


def _run_mlp(u72, rem, m, c, hm, hc, W1, b1, W2, b2, W3, b3):
    bB = 2048
    emb_spec = pl.BlockSpec((bB, E), lambda i: (i, 0))

    def full(shape):
        return pl.BlockSpec(shape, lambda i: tuple(0 for _ in shape))

    w1s = [W1[k * E:(k + 1) * E, :] for k in range(5)]
    return pl.pallas_call(
        _mlp_body,
        grid=(B // bB,),
        in_specs=[pl.BlockSpec((bB, 72), lambda i: (i, 0)),
                  pl.BlockSpec((bB, 1), lambda i: (i, 0))]
        + [emb_spec] * 4 + [full((E, 200))] * 5
        + [full((1, 200)), full((200, 80)), full((1, 80)),
           full((80, 1)), full((1, 1))],
        out_specs=pl.BlockSpec((bB, 1), lambda i: (i, 0)),
        out_shape=jax.ShapeDtypeStruct((B, 1), jnp.float32),
    )(u72, rem, m, c, hm, hc, *w1s, b1.reshape(1, 200), W2, b2.reshape(1, 80),
      W3, b3.reshape(1, 1))


def kernel(uid_batch_ph, mid_batch_ph, mid_his_batch_ph, cat_batch_ph,
           cat_his_batch_ph, mask, seq_len_ph, target_ph, lr, cat_id_1,
           cat_id_2, sim_target, uid_table, mid_table, cat_table,
           W1, b1, W2, b2, W3, b3):
    i32 = jnp.int32
    f32 = jnp.float32
    uid = uid_batch_ph.astype(i32)
    uid_g = (uid >> 2).reshape(NW, RPW)
    uid_r = (18 * (uid & 3)).reshape(B, 1)
    mid_i = mid_batch_ph.astype(i32).reshape(NW, RPW)
    cat_i = cat_batch_ph.astype(i32).reshape(NW, RPW)
    id1_i = cat_id_1.astype(i32).reshape(NW, RPW)
    id2_i = cat_id_2.astype(i32).reshape(NW, RPW)
    mid_h = mid_his_batch_ph.astype(i32).reshape(NW, NBLK, UPB, UENT)
    cat_h = cat_his_batch_ph.astype(i32).reshape(NW, NBLK, UPB, UENT)

    one = 1.0 + 0.0 * lr[0]
    pad = [(0, 0), (0, EP - E)]
    uid72 = (uid_table * one).reshape(-1, 72)
    mid24 = jnp.pad(mid_table, pad) * one
    cat24 = jnp.pad(cat_table, pad) * one

    out_flat = jax.ShapeDtypeStruct((NW, FLAT), f32)
    sc = pl.kernel(
        _sc_body,
        out_type=[jax.ShapeDtypeStruct((NW, RPW, 72), f32)] + [out_flat] * 5,
        mesh=plsc.VectorSubcoreMesh(core_axis_name="c", subcore_axis_name="s",
                                    num_cores=NC, num_subcores=NS),
        scratch_types=[
            pltpu.VMEM((RPW, 72), f32),
            pltpu.VMEM((RPW, EP), f32),
            pltpu.VMEM((RPW, EP), f32),
            pltpu.VMEM((RPW,), i32),
            pltpu.VMEM((UPB, UENT), i32),
            pltpu.VMEM((2, UENT, EP), f32),
            pltpu.VMEM((FLAT + 16,), f32),
            pltpu.SemaphoreType.DMA,
        ],
        compiler_params=pltpu.CompilerParams(use_tc_tiling_on_sc=False),
    )
    uid_e, mid_e, cat_e, sim, hmid, hcat = sc(
        uid72, mid24, cat24,
        uid_g, mid_i, cat_i, id1_i, id2_i, mid_h, cat_h)

    y = _run_mlp(uid_e.reshape(B, 72), uid_r, mid_e.reshape(B, E),
                 cat_e.reshape(B, E), hmid.reshape(B, E), hcat.reshape(B, E),
                 W1, b1, W2, b2, W3, b3)
    return (y, sim.reshape(B, E))


# speedup vs baseline: 1.0598x; 1.0598x over previous
import jax
import jax.numpy as jnp
from jax import lax
from jax.experimental import pallas as pl
from jax.experimental.pallas import tpu as pltpu
from jax.experimental.pallas import tpu_sc as plsc

B = 16384
L = 200
E = 18
EP = 24
NC = 2
NS = 16
NW = NC * NS
RPW = B // NW
UNIT = 4
UENT = UNIT * L
BLK = 64
UPB = BLK // UNIT
NBLK = RPW // BLK
SH = 256
FLAT = RPW * E


def _phase_uid(w, uid72, uid_g, uid_out, ubuf, remv, sem):
    pltpu.sync_copy(uid_g.at[w], remv)
    pltpu.async_copy(uid72.at[remv], ubuf, sem).wait()
    pltpu.sync_copy(ubuf, uid_out.at[w])


def _phase_singles(w, mid_t, cat_t, mid_idx, cat_idx, mid_out, cat_out,
                   sbuf_a, remv, acc, sem):
    def pack_single(src):
        def row(r, carry):
            v0 = src[r, pl.ds(0, 16)]
            v1 = src[r, pl.ds(2, 16)]
            acc[pl.ds(18 * r, 16)] = v0
            acc[pl.ds(18 * r + 2, 16)] = v1
            return carry
        lax.fori_loop(0, RPW, row, 0)

    for tbl, idx_hbm, out in ((mid_t, mid_idx, mid_out),
                              (cat_t, cat_idx, cat_out)):
        pltpu.sync_copy(idx_hbm.at[w], remv)
        pltpu.async_copy(tbl.at[remv], sbuf_a, sem).wait()
        pack_single(sbuf_a)
        pltpu.sync_copy(acc.at[pl.ds(0, FLAT)], out.at[w])


def _phase_sim(w, cat_t, id1_idx, id2_idx, sim_out,
               sbuf_a, sbuf_b, remv, idxblk, acc, sem):
    pltpu.sync_copy(id1_idx.at[w], remv)
    cp1 = pltpu.async_copy(cat_t.at[remv], sbuf_a, sem)
    pltpu.sync_copy(id2_idx.at[w], idxblk.at[0, pl.ds(0, RPW)])
    cp2 = pltpu.async_copy(cat_t.at[idxblk.at[0, pl.ds(0, RPW)]], sbuf_b, sem)
    cp1.wait()
    cp2.wait()

    def sim_row(r, carry):
        a0 = sbuf_a[r, pl.ds(0, 16)]
        b0 = sbuf_b[r, pl.ds(0, 16)]
        a1 = sbuf_a[r, pl.ds(2, 16)]
        b1 = sbuf_b[r, pl.ds(2, 16)]
        acc[pl.ds(18 * r, 16)] = a0 + b0 + a0 * b0
        acc[pl.ds(18 * r + 2, 16)] = a1 + b1 + a1 * b1
        return carry

    lax.fori_loop(0, RPW, sim_row, 0)
    pltpu.sync_copy(acc.at[pl.ds(0, FLAT)], sim_out.at[w])


def _phase_his(w, mid_t, cat_t, mid_his, cat_his, hmid_out, hcat_out,
               idxblk, hrows, acc, sem, sem2):
    z16 = jnp.zeros((16,), jnp.float32)
    sems = (sem, sem2)
    for tbl, his_hbm, out in ((mid_t, mid_his, hmid_out),
                              (cat_t, cat_his, hcat_out)):
        def block_body(blk, bcarry, tbl=tbl, his_hbm=his_hbm):
            pltpu.sync_copy(his_hbm.at[w, blk], idxblk)
            pltpu.async_copy(tbl.at[idxblk.at[0]], hrows.at[0], sems[0])

            def unit_pair(u2, carry, tbl=tbl, blk=blk):
                for p in (0, 1):
                    u = 2 * u2 + p

                    @pl.when(u + 1 < UPB)
                    def _(tbl=tbl, u=u, p=p):
                        pltpu.async_copy(tbl.at[idxblk.at[u + 1]],
                                         hrows.at[1 - p], sems[1 - p])

                    pltpu.make_async_copy(tbl.at[idxblk.at[u]],
                                          hrows.at[p], sems[p]).wait()

                    for q in range(UNIT):
                        def accum(l8, ab, p=p, q=q):
                            a, b = ab
                            base = q * L + l8 * 8
                            lo = [hrows[p, base + i, pl.ds(0, 16)]
                                  for i in range(8)]
                            hi = [hrows[p, base + i, pl.ds(8, 16)]
                                  for i in range(8)]
                            lo = [lo[0] + lo[1], lo[2] + lo[3],
                                  lo[4] + lo[5], lo[6] + lo[7]]
                            hi = [hi[0] + hi[1], hi[2] + hi[3],
                                  hi[4] + hi[5], hi[6] + hi[7]]
                            lo = [lo[0] + lo[1], lo[2] + lo[3]]
                            hi = [hi[0] + hi[1], hi[2] + hi[3]]
                            return (a + (lo[0] + lo[1]),
                                    b + (hi[0] + hi[1]))
                        a, b = lax.fori_loop(0, L // 8, accum, (z16, z16))
                        r = blk * BLK + u * UNIT + q
                        acc[pl.ds(18 * r, 16)] = a
                        acc[pl.ds(18 * r + 8, 16)] = b
                return carry

            lax.fori_loop(0, UPB // 2, unit_pair, 0)
            return bcarry

        lax.fori_loop(0, NBLK, block_body, 0)
        pltpu.sync_copy(acc.at[pl.ds(0, FLAT)], out.at[w])


def _sc_body(uid72, mid_t, cat_t,
             uid_g, mid_idx, cat_idx, id1_idx, id2_idx,
             mid_his, cat_his,
             uid_out, mid_out, cat_out, sim_out, hmid_out, hcat_out,
             ubuf, sbuf_a, sbuf_b, remv, idxblk, hrows, acc, sem, sem2):
    c = lax.axis_index("c")
    s = lax.axis_index("s")
    w = s * NC + c
    _phase_uid(w, uid72, uid_g, uid_out, ubuf, remv, sem)
    _phase_singles(w, mid_t, cat_t, mid_idx, cat_idx, mid_out, cat_out,
                   sbuf_a, remv, acc, sem)
    _phase_sim(w, cat_t, id1_idx, id2_idx, sim_out,
               sbuf_a, sbuf_b, remv, idxblk, acc, sem)
    _phase_his(w, mid_t, cat_t, mid_his, cat_his, hmid_out, hcat_out,
               idxblk, hrows, acc, sem, sem2)


def _mlp_body(u72_ref, rem_ref, m_ref, c_ref, hm_ref, hc_ref,
              w1u, w1m, w1c, w1hm, w1hc, b1, w2, b2, w3, b3, y_ref):
    f32 = jnp.float32
    u72 = u72_ref[...]
    rem = rem_ref[...]
    u = u72[:, 0:18]
    for cshift in (18, 36, 54):
        u = jnp.where(rem == cshift, u72[:, cshift:cshift + 18], u)
    h = (jnp.dot(u, w1u[...], preferred_element_type=f32)
         + jnp.dot(m_ref[...], w1m[...], preferred_element_type=f32)
         + jnp.dot(c_ref[...], w1c[...], preferred_element_type=f32)
         + jnp.dot(hm_ref[...], w1hm[...], preferred_element_type=f32)
         + jnp.dot(hc_ref[...], w1hc[...], preferred_element_type=f32)
         + b1[...])
    h = jnp.maximum(h, 0.0)
    h = jnp.dot(h, w2[...], preferred_element_type=f32) + b2[...]
    h = jnp.maximum(h, 0.0)
    y_ref[...] = jnp.dot(h, w3[...], preferred_element_type=f32) + b3[...]


def _run_mlp(u72, rem, m, c, hm, hc, W1, b1, W2, b2, W3, b3):
    bB = 2048
    emb_spec = pl.BlockSpec((bB, E), lambda i: (i, 0))

    def full(shape):
        return pl.BlockSpec(shape, lambda i: tuple(0 for _ in shape))

    w1s = [W1[k * E:(k + 1) * E, :] for k in range(5)]
    return pl.pallas_call(
        _mlp_body,
        grid=(B // bB,),
        in_specs=[pl.BlockSpec((bB, 72), lambda i: (i, 0)),
                  pl.BlockSpec((bB, 1), lambda i: (i, 0))]
        + [emb_spec] * 4 + [full((E, 200))] * 5
        + [full((1, 200)), full((200, 80)), full((1, 80)),
           full((80, 1)), full((1, 1))],
        out_specs=pl.BlockSpec((bB, 1), lambda i: (i, 0)),
        out_shape=jax.ShapeDtypeStruct((B, 1), jnp.float32),
    )(u72, rem, m, c, hm, hc, *w1s, b1.reshape(1, 200), W2, b2.reshape(1, 80),
      W3, b3.reshape(1, 1))


def kernel(uid_batch_ph, mid_batch_ph, mid_his_batch_ph, cat_batch_ph,
           cat_his_batch_ph, mask, seq_len_ph, target_ph, lr, cat_id_1,
           cat_id_2, sim_target, uid_table, mid_table, cat_table,
           W1, b1, W2, b2, W3, b3):
    i32 = jnp.int32
    f32 = jnp.float32
    uid = uid_batch_ph.astype(i32)
    uid_g = (uid >> 2).reshape(NW, RPW)
    uid_r = (18 * (uid & 3)).reshape(B, 1)
    mid_i = mid_batch_ph.astype(i32).reshape(NW, RPW)
    cat_i = cat_batch_ph.astype(i32).reshape(NW, RPW)
    id1_i = cat_id_1.astype(i32).reshape(NW, RPW)
    id2_i = cat_id_2.astype(i32).reshape(NW, RPW)
    mid_h = mid_his_batch_ph.astype(i32).reshape(NW, NBLK, UPB, UENT)
    cat_h = cat_his_batch_ph.astype(i32).reshape(NW, NBLK, UPB, UENT)

    pad = [(0, 0), (0, EP - E)]
    uid72 = uid_table.reshape(-1, 72)
    mid24 = jnp.pad(mid_table, pad)
    cat24 = jnp.pad(cat_table, pad)

    out_flat = jax.ShapeDtypeStruct((NW, FLAT), f32)
    sc = pl.kernel(
        _sc_body,
        out_type=[jax.ShapeDtypeStruct((NW, RPW, 72), f32)] + [out_flat] * 5,
        mesh=plsc.VectorSubcoreMesh(core_axis_name="c", subcore_axis_name="s",
                                    num_cores=NC, num_subcores=NS),
        scratch_types=[
            pltpu.VMEM((RPW, 72), f32),
            pltpu.VMEM((RPW, EP), f32),
            pltpu.VMEM((RPW, EP), f32),
            pltpu.VMEM((RPW,), i32),
            pltpu.VMEM((UPB, UENT), i32),
            pltpu.VMEM((2, UENT, EP), f32),
            pltpu.VMEM((FLAT + 16,), f32),
            pltpu.SemaphoreType.DMA,
            pltpu.SemaphoreType.DMA,
        ],
        compiler_params=pltpu.CompilerParams(use_tc_tiling_on_sc=False),
    )
    uid_e, mid_e, cat_e, sim, hmid, hcat = sc(
        uid72, mid24, cat24,
        uid_g, mid_i, cat_i, id1_i, id2_i, mid_h, cat_h)

    y = _run_mlp(uid_e.reshape(B, 72), uid_r, mid_e.reshape(B, E),
                 cat_e.reshape(B, E), hmid.reshape(B, E), hcat.reshape(B, E),
                 W1, b1, W2, b2, W3, b3)
    return (y, sim.reshape(B, E))
